# baseline (speedup 1.0000x reference)
import functools

import jax
import jax.numpy as jnp
from jax import lax
from jax.experimental import pallas as pl
from jax.experimental.pallas import tpu as pltpu
from jax.experimental.pallas import tpu_sc as plsc

VOCAB = 100000
EMBEDDING_DIM = 128
MINIBATCH = 4096
NGRAM = 5
HDIM = 2048
TAGS = 1000

TOTAL_ROWS = MINIBATCH * NGRAM

try:
    _info = plsc.get_sparse_core_info()
    _NC, _NS = _info.num_cores, _info.num_subcores
except Exception:
    _NC, _NS = 2, 16
_NW = _NC * _NS
_ROWS_PER_W = TOTAL_ROWS // _NW
_IDX_CHUNK = 128
_N_CHUNKS = _ROWS_PER_W // _IDX_CHUNK


def _make_gather():
    mesh = plsc.VectorSubcoreMesh(core_axis_name="c", subcore_axis_name="s")

    @functools.partial(
        pl.kernel,
        mesh=mesh,
        out_type=jax.ShapeDtypeStruct((TOTAL_ROWS, EMBEDDING_DIM), jnp.float32),
        scratch_types=[
            pltpu.VMEM((_N_CHUNKS, _IDX_CHUNK), jnp.int32),
            pltpu.VMEM((_ROWS_PER_W, EMBEDDING_DIM), jnp.float32),
            pltpu.SemaphoreType.DMA,
        ],
    )
    def gather_k(idx_hbm, table_hbm, out_hbm, idx_v, rows_v, sem):
        wid = lax.axis_index("s") * _NC + lax.axis_index("c")
        base = wid * _ROWS_PER_W
        pltpu.sync_copy(idx_hbm.at[wid], idx_v)
        copies = []
        for j in range(_N_CHUNKS):
            copies.append(
                pltpu.async_copy(
                    table_hbm.at[idx_v.at[j]],
                    rows_v.at[pl.ds(j * _IDX_CHUNK, _IDX_CHUNK)],
                    sem,
                )
            )
        for c in copies:
            c.wait()
        pltpu.sync_copy(rows_v, out_hbm.at[pl.ds(base, _ROWS_PER_W)])

    return gather_k


_gather_cache = []


def _get_gather():
    if not _gather_cache:
        _gather_cache.append(_make_gather())
    return _gather_cache[0]


def _mlp_body(x_ref, w1_ref, b1_ref, w2_ref, b2_ref, o_ref):
    x = x_ref[...].astype(jnp.bfloat16)
    h = jnp.tanh(
        jnp.dot(x, w1_ref[...], preferred_element_type=jnp.float32) + b1_ref[...]
    )
    o_ref[...] = (
        jnp.dot(h.astype(jnp.bfloat16), w2_ref[...],
                preferred_element_type=jnp.float32)
        + b2_ref[...]
    )


_BM = 512


def _mlp(x, W1, b1, W2, b2):
    grid = (MINIBATCH // _BM,)
    return pl.pallas_call(
        _mlp_body,
        grid=grid,
        in_specs=[
            pl.BlockSpec((_BM, NGRAM * EMBEDDING_DIM), lambda i: (i, 0)),
            pl.BlockSpec((NGRAM * EMBEDDING_DIM, HDIM), lambda i: (0, 0)),
            pl.BlockSpec((1, HDIM), lambda i: (0, 0)),
            pl.BlockSpec((HDIM, TAGS), lambda i: (0, 0)),
            pl.BlockSpec((1, TAGS), lambda i: (0, 0)),
        ],
        out_specs=pl.BlockSpec((_BM, TAGS), lambda i: (i, 0)),
        out_shape=jax.ShapeDtypeStruct((MINIBATCH, TAGS), jnp.float32),
    )(x, W1.astype(jnp.bfloat16), b1.reshape(1, HDIM),
      W2.astype(jnp.bfloat16), b2.reshape(1, TAGS))


def kernel(emb, W1, b1, W2, b2, inputs):
    idx = inputs.astype(jnp.int32).reshape(_NW, _N_CHUNKS, _IDX_CHUNK)
    rows = _get_gather()(idx, emb)
    x = rows.reshape(MINIBATCH, NGRAM * EMBEDDING_DIM)
    return _mlp(x, W1, b1, W2, b2)


# speedup vs baseline: 1.0532x; 1.0532x over previous
import functools

import jax
import jax.numpy as jnp
from jax import lax
from jax.experimental import pallas as pl
from jax.experimental.pallas import tpu as pltpu
from jax.experimental.pallas import tpu_sc as plsc

VOCAB = 100000
EMBEDDING_DIM = 128
MINIBATCH = 4096
NGRAM = 5
HDIM = 2048
TAGS = 1000

TOTAL_ROWS = MINIBATCH * NGRAM

try:
    _info = plsc.get_sparse_core_info()
    _NC, _NS = _info.num_cores, _info.num_subcores
except Exception:
    _NC, _NS = 2, 16
_NW = _NC * _NS
_BATCH_PER_W = MINIBATCH // _NW
_IDX_CHUNK = _BATCH_PER_W


def _make_gather():
    mesh = plsc.VectorSubcoreMesh(core_axis_name="c", subcore_axis_name="s")

    @functools.partial(
        pl.kernel,
        mesh=mesh,
        out_type=jax.ShapeDtypeStruct((NGRAM, MINIBATCH, EMBEDDING_DIM),
                                      jnp.float32),
        scratch_types=[
            pltpu.VMEM((NGRAM, _IDX_CHUNK), jnp.int32),
            pltpu.VMEM((NGRAM * _BATCH_PER_W, EMBEDDING_DIM), jnp.float32),
            pltpu.SemaphoreType.DMA,
        ],
    )
    def gather_k(idx_hbm, table_hbm, out_hbm, idx_v, rows_v, sem):
        wid = lax.axis_index("s") * _NC + lax.axis_index("c")
        base = wid * _BATCH_PER_W
        pltpu.sync_copy(idx_hbm.at[wid], idx_v)
        copies = []
        for j in range(NGRAM):
            copies.append(
                pltpu.async_copy(
                    table_hbm.at[idx_v.at[j]],
                    rows_v.at[pl.ds(j * _BATCH_PER_W, _BATCH_PER_W)],
                    sem,
                )
            )
        for c in copies:
            c.wait()
        for j in range(NGRAM):
            pltpu.sync_copy(
                rows_v.at[pl.ds(j * _BATCH_PER_W, _BATCH_PER_W)],
                out_hbm.at[j, pl.ds(base, _BATCH_PER_W)],
            )

    return gather_k


_gather_cache = []


def _get_gather():
    if not _gather_cache:
        _gather_cache.append(_make_gather())
    return _gather_cache[0]


def _mlp_body(x_ref, w1_ref, b1_ref, w2_ref, b2_ref, o_ref):
    acc = jnp.dot(
        x_ref[0].astype(jnp.bfloat16), w1_ref[0],
        preferred_element_type=jnp.float32,
    )
    for j in range(1, NGRAM):
        acc += jnp.dot(
            x_ref[j].astype(jnp.bfloat16), w1_ref[j],
            preferred_element_type=jnp.float32,
        )
    h = jnp.tanh(acc + b1_ref[...])
    o_ref[...] = (
        jnp.dot(h.astype(jnp.bfloat16), w2_ref[...],
                preferred_element_type=jnp.float32)
        + b2_ref[...]
    )


_BM = 512


def _mlp(x, W1, b1, W2, b2):
    grid = (MINIBATCH // _BM,)
    return pl.pallas_call(
        _mlp_body,
        grid=grid,
        in_specs=[
            pl.BlockSpec((NGRAM, _BM, EMBEDDING_DIM), lambda i: (0, i, 0)),
            pl.BlockSpec((NGRAM, EMBEDDING_DIM, HDIM), lambda i: (0, 0, 0)),
            pl.BlockSpec((1, HDIM), lambda i: (0, 0)),
            pl.BlockSpec((HDIM, TAGS), lambda i: (0, 0)),
            pl.BlockSpec((1, TAGS), lambda i: (0, 0)),
        ],
        out_specs=pl.BlockSpec((_BM, TAGS), lambda i: (i, 0)),
        out_shape=jax.ShapeDtypeStruct((MINIBATCH, TAGS), jnp.float32),
    )(x, W1, b1.reshape(1, HDIM), W2, b2.reshape(1, TAGS))


def kernel(emb, W1, b1, W2, b2, inputs):
    idx = (
        inputs.astype(jnp.int32)
        .reshape(_NW, _BATCH_PER_W, NGRAM)
        .transpose(0, 2, 1)
    )
    x = _get_gather()(idx, emb)
    w1 = W1.astype(jnp.bfloat16).reshape(NGRAM, EMBEDDING_DIM, HDIM)
    return _mlp(x, w1, b1, W2.astype(jnp.bfloat16), b2)


# speedup vs baseline: 1.4315x; 1.3592x over previous
import functools

import jax
import jax.numpy as jnp
from jax import lax
from jax.experimental import pallas as pl
from jax.experimental.pallas import tpu as pltpu
from jax.experimental.pallas import tpu_sc as plsc

VOCAB = 100000
EMBEDDING_DIM = 128
MINIBATCH = 4096
NGRAM = 5
HDIM = 2048
TAGS = 1000

TOTAL_ROWS = MINIBATCH * NGRAM

try:
    _info = plsc.get_sparse_core_info()
    _NC, _NS = _info.num_cores, _info.num_subcores
except Exception:
    _NC, _NS = 2, 16
_NW = _NC * _NS
_BATCH_PER_W = MINIBATCH // _NW
_IDX_CHUNK = _BATCH_PER_W


def _make_gather():
    mesh = plsc.VectorSubcoreMesh(core_axis_name="c", subcore_axis_name="s")

    @functools.partial(
        pl.kernel,
        mesh=mesh,
        out_type=jax.ShapeDtypeStruct((NGRAM, MINIBATCH, EMBEDDING_DIM),
                                      jnp.float32),
        scratch_types=[
            pltpu.VMEM((NGRAM, _IDX_CHUNK), jnp.int32),
            pltpu.VMEM((NGRAM * _BATCH_PER_W, EMBEDDING_DIM), jnp.float32),
            pltpu.SemaphoreType.DMA,
        ],
    )
    def gather_k(idx_hbm, table_hbm, out_hbm, idx_v, rows_v, sem):
        wid = lax.axis_index("s") * _NC + lax.axis_index("c")
        base = wid * _BATCH_PER_W
        pltpu.sync_copy(idx_hbm.at[wid], idx_v)
        copies = []
        for j in range(NGRAM):
            copies.append(
                pltpu.async_copy(
                    table_hbm.at[idx_v.at[j]],
                    rows_v.at[pl.ds(j * _BATCH_PER_W, _BATCH_PER_W)],
                    sem,
                )
            )
        for c in copies:
            c.wait()
        for j in range(NGRAM):
            pltpu.sync_copy(
                rows_v.at[pl.ds(j * _BATCH_PER_W, _BATCH_PER_W)],
                out_hbm.at[j, pl.ds(base, _BATCH_PER_W)],
            )

    return gather_k


_gather_cache = []


def _get_gather():
    if not _gather_cache:
        _gather_cache.append(_make_gather())
    return _gather_cache[0]


def _mlp_body(x_ref, w1_ref, b1_ref, w2t_ref, b2t_ref, o_ref):
    xw = jnp.concatenate(
        [x_ref[j].astype(jnp.bfloat16) for j in range(NGRAM)], axis=1
    )
    h = jnp.tanh(
        jnp.dot(xw, w1_ref[...], preferred_element_type=jnp.float32)
        + b1_ref[...]
    )
    o_ref[...] = (
        lax.dot_general(
            w2t_ref[...], h.astype(jnp.bfloat16),
            (((1,), (1,)), ((), ())),
            preferred_element_type=jnp.float32,
        )
        + b2t_ref[...]
    )


_BM = 512


def _mlp(x, W1, b1, W2t, b2):
    grid = (MINIBATCH // _BM,)
    return pl.pallas_call(
        _mlp_body,
        grid=grid,
        in_specs=[
            pl.BlockSpec((NGRAM, _BM, EMBEDDING_DIM), lambda i: (0, i, 0)),
            pl.BlockSpec((NGRAM * EMBEDDING_DIM, HDIM), lambda i: (0, 0)),
            pl.BlockSpec((1, HDIM), lambda i: (0, 0)),
            pl.BlockSpec((TAGS, HDIM), lambda i: (0, 0)),
            pl.BlockSpec((TAGS, 1), lambda i: (0, 0)),
        ],
        out_specs=pl.BlockSpec((TAGS, _BM), lambda i: (0, i)),
        out_shape=jax.ShapeDtypeStruct((TAGS, MINIBATCH), jnp.float32),
    )(x, W1, b1.reshape(1, HDIM), W2t, b2.reshape(TAGS, 1))


def kernel(emb, W1, b1, W2, b2, inputs):
    idx = (
        inputs.astype(jnp.int32)
        .reshape(_NW, _BATCH_PER_W, NGRAM)
        .transpose(0, 2, 1)
    )
    x = _get_gather()(idx, emb)
    w1 = W1.astype(jnp.bfloat16)
    w2t = W2.T.astype(jnp.bfloat16)
    out_t = _mlp(x, w1, b1, w2t, b2)
    return out_t.T
